# baseline (speedup 1.0000x reference)
import functools

import jax
import jax.numpy as jnp
from jax import lax
from jax.experimental import pallas as pl
from jax.experimental.pallas import tpu as pltpu
from jax.experimental.pallas import tpu_sc as plsc

N = 10000
NPAD = 10240
E = 160000
IN_DIM = 128
HIDDEN = 32
OUT_DIM = 16

NC = 2
NS = 16
NW = NC * NS
GROUP = 128
GPW = 40
EPW = GPW * GROUP
EPAD = EPW * NW
RING = 8
DEPTH = 6
ROWS_PER_TILE = NPAD // NS

_mesh = plsc.VectorSubcoreMesh(
    core_axis_name="c", subcore_axis_name="s", num_cores=NC, num_subcores=NS
)
_sc_params = pltpu.CompilerParams(use_tc_tiling_on_sc=False, needs_layout_passes=False)


def _deg_body(col2_hbm, w2_hbm, out_hbm, acc, col40, w40, stage, ssem):
    cid = lax.axis_index("c")
    sid = lax.axis_index("s")
    wid = sid * NC + cid
    r0 = sid * ROWS_PER_TILE
    gb = wid * GPW
    pltpu.sync_copy(col2_hbm.at[pl.ds(gb, GPW)], col40)
    pltpu.sync_copy(w2_hbm.at[pl.ds(gb, GPW)], w40)

    z = jnp.zeros((16,), jnp.float32)

    def zrow(r, c):
        stage[pl.ds(r * 16, 16)] = z
        return c

    lax.fori_loop(0, ROWS_PER_TILE // 16, zrow, 0)
    pltpu.sync_copy(stage, acc.at[pl.ds(r0, ROWS_PER_TILE)])
    plsc.subcore_barrier()

    def fire(k, c):
        pltpu.async_copy(w40.at[k], acc.at[col40.at[k]], ssem, add=True)
        return c

    lax.fori_loop(0, GPW, fire, 0)

    def drain(k, c):
        pltpu.make_async_copy(w40.at[0], acc.at[col40.at[0]], ssem).wait()
        return c

    lax.fori_loop(0, GPW, drain, 0)
    plsc.subcore_barrier()
    pltpu.sync_copy(acc.at[pl.ds(r0, ROWS_PER_TILE)], stage)
    pltpu.sync_copy(stage, out_hbm.at[cid, pl.ds(r0, ROWS_PER_TILE)])


_deg_kernel = functools.partial(
    pl.kernel,
    out_type=jax.ShapeDtypeStruct((NC, NPAD), jnp.float32),
    mesh=_mesh,
    compiler_params=_sc_params,
    scratch_types=[
        pltpu.VMEM_SHARED((NPAD,), jnp.float32),
        pltpu.VMEM((GPW, GROUP), jnp.int32),
        pltpu.VMEM((GPW, GROUP), jnp.float32),
        pltpu.VMEM((ROWS_PER_TILE,), jnp.float32),
        pltpu.SemaphoreType.DMA,
    ],
)(_deg_body)


def _make_prop(F):
    def body(row2_hbm, col2_hbm, w2_hbm, g_hbm, out_hbm,
             acc, row40, col40, w40, rows, stage, gsem, ssem):
        cid = lax.axis_index("c")
        sid = lax.axis_index("s")
        wid = sid * NC + cid
        r0 = sid * ROWS_PER_TILE
        gb = wid * GPW
        pltpu.sync_copy(row2_hbm.at[pl.ds(gb, GPW)], row40)
        pltpu.sync_copy(col2_hbm.at[pl.ds(gb, GPW)], col40)
        pltpu.sync_copy(w2_hbm.at[pl.ds(gb, GPW)], w40)
        for kp in range(DEPTH):
            pltpu.async_copy(g_hbm.at[row40.at[kp]], rows.at[kp], gsem.at[kp])

        z32 = jnp.zeros((32,), jnp.bfloat16)

        def zrow(r, c):
            stage[r, :] = z32
            return c

        lax.fori_loop(0, ROWS_PER_TILE, zrow, 0)
        pltpu.sync_copy(stage, acc.at[pl.ds(r0, ROWS_PER_TILE)])
        plsc.subcore_barrier()

        def step(k, carry):
            b = lax.rem(k, RING)
            pltpu.make_async_copy(
                g_hbm.at[row40.at[k]], rows.at[b], gsem.at[b]
            ).wait()

            for g16 in range(GROUP // 16):
                wv = w40[k, pl.ds(g16 * 16, 16)]
                for i in range(16):
                    ws = wv[i]
                    e = g16 * 16 + i
                    va, vb = plsc.unpack(rows[b, e, :],
                                         format=plsc.PackFormat.INTERLEAVED)
                    rows[b, e, :] = plsc.pack(
                        va * ws, vb * ws,
                        format=plsc.PackFormat.INTERLEAVED)
            pltpu.async_copy(rows.at[b], acc.at[col40.at[k]], ssem.at[b],
                             add=True)

            @pl.when(k + DEPTH < GPW)
            def _():
                b2 = lax.rem(k + DEPTH, RING)

                @pl.when(k >= RING - DEPTH)
                def _():
                    pltpu.make_async_copy(
                        rows.at[b2], acc.at[col40.at[0]], ssem.at[b2]
                    ).wait()

                pltpu.async_copy(
                    g_hbm.at[row40.at[k + DEPTH]], rows.at[b2], gsem.at[b2]
                )

            return carry

        lax.fori_loop(0, GPW, step, 0)
        for kd in range(GPW - RING, GPW):
            bd = kd % RING
            pltpu.make_async_copy(
                rows.at[bd], acc.at[col40.at[0]], ssem.at[bd]
            ).wait()
        plsc.subcore_barrier()
        pltpu.sync_copy(acc.at[pl.ds(r0, ROWS_PER_TILE)], stage)
        pltpu.sync_copy(stage, out_hbm.at[cid, pl.ds(r0, ROWS_PER_TILE)])

    return functools.partial(
        pl.kernel,
        out_type=jax.ShapeDtypeStruct((NC, NPAD, F), jnp.bfloat16),
        mesh=_mesh,
        compiler_params=_sc_params,
        scratch_types=[
            pltpu.VMEM_SHARED((NPAD, F), jnp.bfloat16),
            pltpu.VMEM((GPW, GROUP), jnp.int32),
            pltpu.VMEM((GPW, GROUP), jnp.int32),
            pltpu.VMEM((GPW, GROUP), jnp.float32),
            pltpu.VMEM((RING, GROUP, F), jnp.bfloat16),
            pltpu.VMEM((ROWS_PER_TILE, F), jnp.bfloat16),
            pltpu.SemaphoreType.DMA((RING,)),
            pltpu.SemaphoreType.DMA((RING,)),
        ],
    )(body)


_prop32 = _make_prop(HIDDEN)


def _dinv_from(degp_ref):
    deg = (degp_ref[0] + degp_ref[1])[:N] + 1.0
    return lax.rsqrt(deg).reshape(N, 1)


def _tc_first(degp, x, W1):
    def body(d_ref, x_ref, w_ref, o_ref):
        dinv = _dinv_from(d_ref)
        h = jnp.dot(x_ref[...], w_ref[...], preferred_element_type=jnp.float32)
        o_ref[...] = (h * dinv).astype(jnp.bfloat16)

    return pl.pallas_call(
        body, out_shape=jax.ShapeDtypeStruct((N, W1.shape[1]), jnp.bfloat16)
    )(degp, x, W1)


def _tc_mid(p, g, degp, b, Wn):
    def body(p_ref, g_ref, d_ref, b_ref, w_ref, o_ref):
        dinv = _dinv_from(d_ref)
        psum = (p_ref[0].astype(jnp.float32) + p_ref[1].astype(jnp.float32))[:N]
        h = (psum + g_ref[...].astype(jnp.float32)) * dinv + b_ref[...]
        h = jnp.maximum(h, 0.0)
        o_ref[...] = (
            jnp.dot(h, w_ref[...], preferred_element_type=jnp.float32) * dinv
        ).astype(jnp.bfloat16)

    return pl.pallas_call(
        body, out_shape=jax.ShapeDtypeStruct((N, Wn.shape[1]), jnp.bfloat16)
    )(p, g, degp, b.reshape(1, -1), Wn)


def _tc_last(p, g, degp, b):
    def body(p_ref, g_ref, d_ref, b_ref, o_ref):
        dinv = _dinv_from(d_ref)
        psum = (p_ref[0].astype(jnp.float32) + p_ref[1].astype(jnp.float32))[:N]
        o_ref[...] = jnp.tanh(
            (psum + g_ref[...].astype(jnp.float32))[:, :OUT_DIM] * dinv
            + b_ref[...]
        )

    return pl.pallas_call(
        body, out_shape=jax.ShapeDtypeStruct((N, OUT_DIM), jnp.float32)
    )(p, g, degp, b.reshape(1, -1))


_GRAM_BR = 400


def _tc_gram(h3, h3t):
    def body(a_ref, bt_ref, o_ref):
        o_ref[...] = jax.nn.sigmoid(
            jnp.dot(a_ref[...], bt_ref[...], preferred_element_type=jnp.float32)
        )

    h3 = h3.astype(jnp.bfloat16)
    h3t = h3t.astype(jnp.bfloat16)

    return pl.pallas_call(
        body,
        grid=(N // _GRAM_BR,),
        in_specs=[
            pl.BlockSpec((_GRAM_BR, OUT_DIM), lambda i: (i, 0)),
            pl.BlockSpec((OUT_DIM, N), lambda i: (0, 0)),
        ],
        out_specs=pl.BlockSpec((_GRAM_BR, N), lambda i: (i, 0)),
        out_shape=jax.ShapeDtypeStruct((N, N), jnp.float32),
    )(h3, h3t)


def kernel(x, edge_index, edge_weight, W1, b1, W2, b2, W3, b3):
    row = edge_index[0]
    col = edge_index[1]
    padw = EPW - E // NW
    padi = (jnp.arange(NW * padw, dtype=jnp.int32) % N).reshape(NW, padw)

    def pack_idx(a):
        return jnp.concatenate(
            [a.reshape(NW, E // NW), padi], axis=1
        ).reshape(NW * GPW, GROUP)

    row2 = pack_idx(row)
    col2 = pack_idx(col)
    w2 = jnp.concatenate(
        [edge_weight.reshape(NW, E // NW),
         jnp.zeros((NW, padw), edge_weight.dtype)],
        axis=1,
    ).reshape(NW * GPW, GROUP)

    degp = _deg_kernel(col2, w2)

    g1 = _tc_first(degp, x, W1)
    p1 = _prop32(row2, col2, w2, g1)
    g2 = _tc_mid(p1, g1, degp, b1, W2)
    p2 = _prop32(row2, col2, w2, g2)
    W3p = jnp.concatenate(
        [W3, jnp.zeros((HIDDEN, HIDDEN - OUT_DIM), W3.dtype)], axis=1
    )
    g3 = _tc_mid(p2, g2, degp, b2, W3p)
    p3 = _prop32(row2, col2, w2, g3)
    h3 = _tc_last(p3, g3, degp, b3)
    return _tc_gram(h3, h3.T)


# speedup vs baseline: 1.0653x; 1.0653x over previous
import functools

import jax
import jax.numpy as jnp
from jax import lax
from jax.experimental import pallas as pl
from jax.experimental.pallas import tpu as pltpu
from jax.experimental.pallas import tpu_sc as plsc

N = 10000
NPAD = 10240
E = 160000
IN_DIM = 128
HIDDEN = 32
OUT_DIM = 16

NC = 2
NS = 16
NW = NC * NS
GROUP = 128
GPW = 40
EPW = GPW * GROUP
EPAD = EPW * NW
RING = 8
DEPTH = 6
ROWS_PER_TILE = NPAD // NS

_mesh = plsc.VectorSubcoreMesh(
    core_axis_name="c", subcore_axis_name="s", num_cores=NC, num_subcores=NS
)
_sc_params = pltpu.CompilerParams(use_tc_tiling_on_sc=False, needs_layout_passes=False)


def _deg_body(col2_hbm, w2_hbm, out_hbm, acc, col40, w40, stage, ssem):
    cid = lax.axis_index("c")
    sid = lax.axis_index("s")
    wid = sid * NC + cid
    r0 = sid * ROWS_PER_TILE
    gb = wid * GPW
    pltpu.sync_copy(col2_hbm.at[pl.ds(gb, GPW)], col40)
    pltpu.sync_copy(w2_hbm.at[pl.ds(gb, GPW)], w40)

    z = jnp.zeros((16,), jnp.float32)

    def zrow(r, c):
        stage[pl.ds(r * 16, 16)] = z
        return c

    lax.fori_loop(0, ROWS_PER_TILE // 16, zrow, 0)
    pltpu.sync_copy(stage, acc.at[pl.ds(r0, ROWS_PER_TILE)])
    plsc.subcore_barrier()

    def fire(k, c):
        pltpu.async_copy(w40.at[k], acc.at[col40.at[k]], ssem, add=True)
        return c

    lax.fori_loop(0, GPW, fire, 0)

    def drain(k, c):
        pltpu.make_async_copy(w40.at[0], acc.at[col40.at[0]], ssem).wait()
        return c

    lax.fori_loop(0, GPW, drain, 0)
    plsc.subcore_barrier()
    pltpu.sync_copy(acc.at[pl.ds(r0, ROWS_PER_TILE)], stage)
    pltpu.sync_copy(stage, out_hbm.at[cid, pl.ds(r0, ROWS_PER_TILE)])


_deg_kernel = functools.partial(
    pl.kernel,
    out_type=jax.ShapeDtypeStruct((NC, NPAD), jnp.float32),
    mesh=_mesh,
    compiler_params=_sc_params,
    scratch_types=[
        pltpu.VMEM_SHARED((NPAD,), jnp.float32),
        pltpu.VMEM((GPW, GROUP), jnp.int32),
        pltpu.VMEM((GPW, GROUP), jnp.float32),
        pltpu.VMEM((ROWS_PER_TILE,), jnp.float32),
        pltpu.SemaphoreType.DMA,
    ],
)(_deg_body)


def _make_prop(F):
    def body(row2_hbm, col2_hbm, w2_hbm, g_hbm, out_hbm,
             acc, row40, col40, w40, rows, stage, stage2, gsem, ssem):
        cid = lax.axis_index("c")
        sid = lax.axis_index("s")
        wid = sid * NC + cid
        r0 = sid * ROWS_PER_TILE
        gb = wid * GPW
        c1 = pltpu.async_copy(row2_hbm.at[pl.ds(gb, GPW)], row40, gsem.at[6])
        c2 = pltpu.async_copy(col2_hbm.at[pl.ds(gb, GPW)], col40, gsem.at[7])
        c3 = pltpu.async_copy(w2_hbm.at[pl.ds(gb, GPW)], w40, gsem.at[6])
        c1.wait()
        c2.wait()
        c3.wait()
        for kp in range(DEPTH):
            pltpu.async_copy(g_hbm.at[row40.at[kp]], rows.at[kp], gsem.at[kp])

        z32 = jnp.zeros((32,), jnp.bfloat16)

        def zrow(r, c):
            stage[r, :] = z32
            return c

        lax.fori_loop(0, ROWS_PER_TILE, zrow, 0)
        pltpu.sync_copy(stage, acc.at[pl.ds(r0, ROWS_PER_TILE)])
        plsc.subcore_barrier()

        def step(k, carry):
            b = lax.rem(k, RING)
            pltpu.make_async_copy(
                g_hbm.at[row40.at[k]], rows.at[b], gsem.at[b]
            ).wait()

            for g16 in range(GROUP // 16):
                wv = w40[k, pl.ds(g16 * 16, 16)]
                for i in range(16):
                    ws = wv[i]
                    e = g16 * 16 + i
                    va, vb = plsc.unpack(rows[b, e, :],
                                         format=plsc.PackFormat.INTERLEAVED)
                    rows[b, e, :] = plsc.pack(
                        va * ws, vb * ws,
                        format=plsc.PackFormat.INTERLEAVED)
            pltpu.async_copy(rows.at[b], acc.at[col40.at[k]], ssem.at[b],
                             add=True)

            @pl.when(k + DEPTH < GPW)
            def _():
                b2 = lax.rem(k + DEPTH, RING)

                @pl.when(k >= RING - DEPTH)
                def _():
                    pltpu.make_async_copy(
                        rows.at[b2], acc.at[col40.at[0]], ssem.at[b2]
                    ).wait()

                pltpu.async_copy(
                    g_hbm.at[row40.at[k + DEPTH]], rows.at[b2], gsem.at[b2]
                )

            return carry

        lax.fori_loop(0, GPW, step, 0)
        for kd in range(GPW - RING, GPW):
            bd = kd % RING
            pltpu.make_async_copy(
                rows.at[bd], acc.at[col40.at[0]], ssem.at[bd]
            ).wait()
        plsc.subcore_barrier()
        pltpu.sync_copy(acc.at[pl.ds(r0, ROWS_PER_TILE)], stage)
        prt = ROWS_PER_TILE * F // 128

        def repack(r, c):
            for a in range(128 // F):
                stage2[r, pl.ds(F * a, F)] = stage[(128 // F) * r + a, :]
            return c

        lax.fori_loop(0, prt, repack, 0)
        pltpu.sync_copy(stage2, out_hbm.at[cid, pl.ds(sid * prt, prt)])

    return functools.partial(
        pl.kernel,
        out_type=jax.ShapeDtypeStruct((NC, NPAD * F // 128, 128),
                                      jnp.bfloat16),
        mesh=_mesh,
        compiler_params=_sc_params,
        scratch_types=[
            pltpu.VMEM_SHARED((NPAD, F), jnp.bfloat16),
            pltpu.VMEM((GPW, GROUP), jnp.int32),
            pltpu.VMEM((GPW, GROUP), jnp.int32),
            pltpu.VMEM((GPW, GROUP), jnp.float32),
            pltpu.VMEM((RING, GROUP, F), jnp.bfloat16),
            pltpu.VMEM((ROWS_PER_TILE, F), jnp.bfloat16),
            pltpu.VMEM((ROWS_PER_TILE * F // 128, 128), jnp.bfloat16),
            pltpu.SemaphoreType.DMA((RING,)),
            pltpu.SemaphoreType.DMA((RING,)),
        ],
    )(body)


_prop32 = _make_prop(HIDDEN)


NP4 = N // 4


def _tc_dinv2d(degp2d):
    def body(d_ref, o_ref):
        o_ref[...] = lax.rsqrt(d_ref[0] + d_ref[1] + 1.0)

    return pl.pallas_call(
        body, out_shape=jax.ShapeDtypeStruct((NPAD // 128, 128), jnp.float32)
    )(degp2d)


def _tc_first(dinv4, x4, W4):
    def body(v_ref, x_ref, w_ref, o_ref):
        h = jnp.dot(x_ref[...], w_ref[...], preferred_element_type=jnp.float32)
        o_ref[...] = (h * v_ref[...]).astype(jnp.bfloat16)

    return pl.pallas_call(
        body, out_shape=jax.ShapeDtypeStruct((NP4, 128), jnp.bfloat16)
    )(dinv4, x4, W4)


def _tc_mid(p, g, dinv4, b4, W4):
    def body(p_ref, g_ref, v_ref, b_ref, w_ref, o_ref):
        psum = (p_ref[0].astype(jnp.float32)
                + p_ref[1].astype(jnp.float32))[:NP4]
        h = (psum + g_ref[...].astype(jnp.float32)) * v_ref[...] + b_ref[...]
        h = jnp.maximum(h, 0.0)
        o_ref[...] = (
            jnp.dot(h, w_ref[...], preferred_element_type=jnp.float32)
            * v_ref[...]
        ).astype(jnp.bfloat16)

    return pl.pallas_call(
        body, out_shape=jax.ShapeDtypeStruct((NP4, 128), jnp.bfloat16)
    )(p, g, dinv4, b4, W4)


def _tc_last(p, g, degp, b):
    def body(p_ref, g_ref, d_ref, b_ref, o_ref):
        dinv = lax.rsqrt((d_ref[0] + d_ref[1])[:N] + 1.0).reshape(N, 1)
        psum = (p_ref[0].astype(jnp.float32) + p_ref[1].astype(jnp.float32))[:N]
        o_ref[...] = jnp.tanh(
            (psum + g_ref[...].astype(jnp.float32))[:, :OUT_DIM] * dinv
            + b_ref[...]
        )

    return pl.pallas_call(
        body, out_shape=jax.ShapeDtypeStruct((N, OUT_DIM), jnp.float32)
    )(p, g, degp, b.reshape(1, -1))


_GRAM_BR = 400


def _tc_gram(h3, h3t):
    def body(a_ref, bt_ref, o_ref):
        o_ref[...] = jax.nn.sigmoid(
            jnp.dot(a_ref[...], bt_ref[...], preferred_element_type=jnp.float32)
        )


    return pl.pallas_call(
        body,
        grid=(N // _GRAM_BR,),
        in_specs=[
            pl.BlockSpec((_GRAM_BR, OUT_DIM), lambda i: (i, 0)),
            pl.BlockSpec((OUT_DIM, N), lambda i: (0, 0)),
        ],
        out_specs=pl.BlockSpec((_GRAM_BR, N), lambda i: (i, 0)),
        out_shape=jax.ShapeDtypeStruct((N, N), jnp.float32),
    )(h3, h3t)


def kernel(x, edge_index, edge_weight, W1, b1, W2, b2, W3, b3):
    row = edge_index[0]
    col = edge_index[1]
    padw = EPW - E // NW
    padi = (jnp.arange(NW * padw, dtype=jnp.int32) % N).reshape(NW, padw)

    def pack_idx(a):
        return jnp.concatenate(
            [a.reshape(NW, E // NW), padi], axis=1
        ).reshape(NW * GPW, GROUP)

    row2 = pack_idx(row)
    col2 = pack_idx(col)
    w2 = jnp.concatenate(
        [edge_weight.reshape(NW, E // NW),
         jnp.zeros((NW, padw), edge_weight.dtype)],
        axis=1,
    ).reshape(NW * GPW, GROUP)

    degp = _deg_kernel(col2, w2)
    dinv2d = _tc_dinv2d(degp.reshape(NC, NPAD // 128, 128))
    dinv4 = jnp.repeat(
        dinv2d.reshape(NPAD)[:N].reshape(NP4, 4), 32, axis=1
    )

    eye4 = jnp.eye(4, dtype=jnp.float32)
    W3p = jnp.concatenate(
        [W3, jnp.zeros((HIDDEN, HIDDEN - OUT_DIM), W3.dtype)], axis=1
    )
    x4 = x.reshape(NP4, 4 * IN_DIM)

    g1 = _tc_first(dinv4, x4, jnp.kron(eye4, W1))
    p1 = _prop32(row2, col2, w2, g1.reshape(N, HIDDEN))
    g2 = _tc_mid(p1, g1, dinv4, jnp.tile(b1, 4).reshape(1, 128),
                 jnp.kron(eye4, W2))
    p2 = _prop32(row2, col2, w2, g2.reshape(N, HIDDEN))
    g3 = _tc_mid(p2, g2, dinv4, jnp.tile(b2, 4).reshape(1, 128),
                 jnp.kron(eye4, W3p))
    p3 = _prop32(row2, col2, w2, g3.reshape(N, HIDDEN))
    h3 = _tc_last(p3.reshape(NC, NPAD, HIDDEN), g3.reshape(N, HIDDEN),
                  degp, b3)
    return _tc_gram(h3, h3.T)
